# Initial kernel scaffold; baseline (speedup 1.0000x reference)
import jax
import jax.numpy as jnp
from jax.experimental import pallas as pl


def kernel(indices, embed):
    raise NotImplementedError("write your pallas kernel here")


# speedup vs baseline: 1.3662x; 1.3662x over previous
import jax
import jax.numpy as jnp
from jax import lax
from jax.experimental import pallas as pl
from jax.experimental.pallas import tpu as pltpu
from jax.experimental.pallas import tpu_sc as plsc

_NC, _NS = 2, 16
_NW = _NC * _NS

_B = 16384 * 26
_D = 32
_BPW = _B // _NW
_CHUNK = 128
_NCHUNK = _BPW // _CHUNK


def _gather_body(idx_hbm, table_hbm, out_hbm, idx_v, rows_v, sem):
    wid = lax.axis_index("s") * _NC + lax.axis_index("c")
    base = wid * _BPW

    @pl.loop(0, _NCHUNK)
    def _chunk(c):
        off = base + c * _CHUNK
        pltpu.sync_copy(idx_hbm.at[pl.ds(off, _CHUNK)], idx_v)
        pltpu.async_copy(table_hbm.at[idx_v], rows_v, sem).wait()
        pltpu.sync_copy(rows_v, out_hbm.at[pl.ds(off, _CHUNK)])


@jax.jit
def kernel(indices, embed):
    flat = indices.reshape(-1).astype(jnp.int32)
    mesh = plsc.VectorSubcoreMesh(core_axis_name="c", subcore_axis_name="s")
    out = pl.kernel(
        _gather_body,
        out_type=jax.ShapeDtypeStruct((_B, _D), jnp.float32),
        mesh=mesh,
        scratch_types=[
            pltpu.VMEM((_CHUNK,), jnp.int32),
            pltpu.VMEM((_CHUNK, _D), jnp.float32),
            pltpu.SemaphoreType.DMA,
        ],
        compiler_params=pltpu.CompilerParams(use_tc_tiling_on_sc=False),
    )(flat, embed)
    return out.reshape(indices.shape + (_D,))


# speedup vs baseline: 1.5485x; 1.1334x over previous
import jax
import jax.numpy as jnp
from jax import lax
from jax.experimental import pallas as pl
from jax.experimental.pallas import tpu as pltpu
from jax.experimental.pallas import tpu_sc as plsc

_NC, _NS = 2, 16
_NW = _NC * _NS

_B = 16384 * 26
_D = 32
_BPW = _B // _NW
_CHUNK = 1024
_NCHUNK = _BPW // _CHUNK


def _gather_body(idx_hbm, table_hbm, out_hbm, idx_v, rows_v, sem):
    wid = lax.axis_index("s") * _NC + lax.axis_index("c")
    base = wid * _BPW

    @pl.loop(0, _NCHUNK)
    def _chunk(c):
        off = base + c * _CHUNK
        pltpu.sync_copy(idx_hbm.at[pl.ds(off, _CHUNK)], idx_v)
        pltpu.async_copy(table_hbm.at[idx_v], rows_v, sem).wait()
        pltpu.sync_copy(rows_v, out_hbm.at[pl.ds(off, _CHUNK)])


@jax.jit
def kernel(indices, embed):
    flat = indices.reshape(-1).astype(jnp.int32)
    mesh = plsc.VectorSubcoreMesh(core_axis_name="c", subcore_axis_name="s")
    out = pl.kernel(
        _gather_body,
        out_type=jax.ShapeDtypeStruct((_B, _D), jnp.float32),
        mesh=mesh,
        scratch_types=[
            pltpu.VMEM((_CHUNK,), jnp.int32),
            pltpu.VMEM((_CHUNK, _D), jnp.float32),
            pltpu.SemaphoreType.DMA,
        ],
        compiler_params=pltpu.CompilerParams(use_tc_tiling_on_sc=False),
    )(flat, embed)
    return out.reshape(indices.shape + (_D,))


# speedup vs baseline: 1.5758x; 1.0176x over previous
import jax
import jax.numpy as jnp
from jax import lax
from jax.experimental import pallas as pl
from jax.experimental.pallas import tpu as pltpu
from jax.experimental.pallas import tpu_sc as plsc

_NC, _NS = 2, 16
_NW = _NC * _NS

_B = 16384 * 26
_D = 32
_BPW = _B // _NW
_STEP = 832
_NSTEP = _BPW // _STEP


def _gather_body(idx_hbm, table_hbm, out_hbm,
                 idx_v0, idx_v1, rows_v0, rows_v1,
                 sem_i0, sem_i1, sem_g0, sem_g1, sem_o0, sem_o1):
    wid = lax.axis_index("s") * _NC + lax.axis_index("c")
    base = wid * _BPW
    idx_v = (idx_v0, idx_v1)
    rows_v = (rows_v0, rows_v1)
    sem_i = (sem_i0, sem_i1)
    sem_g = (sem_g0, sem_g1)
    sem_o = (sem_o0, sem_o1)

    def off(s):
        return base + s * _STEP

    pltpu.sync_copy(idx_hbm.at[pl.ds(off(0), _STEP)], idx_v[0])
    pltpu.async_copy(table_hbm.at[idx_v[0]], rows_v[0], sem_g[0])
    pltpu.async_copy(idx_hbm.at[pl.ds(off(1), _STEP)], idx_v[1], sem_i[1])
    pltpu.make_async_copy(table_hbm.at[idx_v[0]], rows_v[0], sem_g[0]).wait()
    pltpu.async_copy(idx_hbm.at[pl.ds(off(2), _STEP)], idx_v[0], sem_i[0])
    pltpu.async_copy(rows_v[0], out_hbm.at[pl.ds(off(0), _STEP)], sem_o[0])
    pltpu.make_async_copy(idx_hbm.at[pl.ds(off(1), _STEP)], idx_v[1],
                          sem_i[1]).wait()
    pltpu.async_copy(table_hbm.at[idx_v[1]], rows_v[1], sem_g[1])

    def sub_body(s, b):
        o = b ^ 1
        pltpu.make_async_copy(
            rows_v[b], out_hbm.at[pl.ds(off(s - 2), _STEP)], sem_o[b]).wait()
        pltpu.make_async_copy(
            idx_hbm.at[pl.ds(off(s), _STEP)], idx_v[b], sem_i[b]).wait()
        pltpu.async_copy(table_hbm.at[idx_v[b]], rows_v[b], sem_g[b])
        pltpu.make_async_copy(table_hbm.at[idx_v[o]], rows_v[o],
                              sem_g[o]).wait()

        @pl.when(s + 1 < _NSTEP)
        def _():
            pltpu.async_copy(idx_hbm.at[pl.ds(off(s + 1), _STEP)], idx_v[o],
                             sem_i[o])

        pltpu.async_copy(rows_v[o], out_hbm.at[pl.ds(off(s - 1), _STEP)],
                         sem_o[o])

    @pl.loop(2, _NSTEP, step=2)
    def _steps(s):
        sub_body(s, 0)
        sub_body(s + 1, 1)

    last = _NSTEP - 1
    pltpu.make_async_copy(table_hbm.at[idx_v[1]], rows_v[1], sem_g[1]).wait()
    pltpu.make_async_copy(
        rows_v[0], out_hbm.at[pl.ds(off(last - 1), _STEP)], sem_o[0]).wait()
    pltpu.async_copy(rows_v[1], out_hbm.at[pl.ds(off(last), _STEP)], sem_o[1])
    pltpu.make_async_copy(
        rows_v[1], out_hbm.at[pl.ds(off(last), _STEP)], sem_o[1]).wait()


@jax.jit
def kernel(indices, embed):
    flat = indices.reshape(-1).astype(jnp.int32)
    mesh = plsc.VectorSubcoreMesh(core_axis_name="c", subcore_axis_name="s")
    out = pl.kernel(
        _gather_body,
        out_type=jax.ShapeDtypeStruct((_B, _D), jnp.float32),
        mesh=mesh,
        scratch_types=[
            pltpu.VMEM((_STEP,), jnp.int32),
            pltpu.VMEM((_STEP,), jnp.int32),
            pltpu.VMEM((_STEP, _D), jnp.float32),
            pltpu.VMEM((_STEP, _D), jnp.float32),
            pltpu.SemaphoreType.DMA,
            pltpu.SemaphoreType.DMA,
            pltpu.SemaphoreType.DMA,
            pltpu.SemaphoreType.DMA,
            pltpu.SemaphoreType.DMA,
            pltpu.SemaphoreType.DMA,
        ],
        compiler_params=pltpu.CompilerParams(use_tc_tiling_on_sc=False),
    )(flat, embed)
    return out.reshape(indices.shape + (_D,))
